# Initial kernel scaffold; baseline (speedup 1.0000x reference)
import jax
import jax.numpy as jnp
from jax.experimental import pallas as pl


def kernel(x, action_emb):
    raise NotImplementedError("write your pallas kernel here")


# speedup vs baseline: 20.1226x; 20.1226x over previous
import functools

import jax
import jax.numpy as jnp
from jax import lax
from jax.experimental import pallas as pl
from jax.experimental.pallas import tpu as pltpu
from jax.experimental.pallas import tpu_sc as plsc

_B, _H, _D = 16384, 200, 32
_NC, _NS = 2, 16
_NW = _NC * _NS
_RPW = _B // _NW
_IB = 32
_NBLK = _RPW // _IB
_G0 = 128
_G1 = _H - _G0
_UNROLL = 8


def _emb_pool_body(x_hbm, emb_hbm, out_hbm, idx_v, rows_v, out_v, sem_g):
    wid = lax.axis_index("s") * _NC + lax.axis_index("c")
    row0 = wid * _RPW

    def blk_body(b, carry):
        base = row0 + b * _IB
        pltpu.sync_copy(x_hbm.at[pl.ds(base, _IB), :], idx_v)

        def row_body(rl, carry):
            c0 = pltpu.async_copy(
                emb_hbm.at[idx_v.at[rl, pl.ds(0, _G0)]],
                rows_v.at[pl.ds(0, _G0), :], sem_g)
            c1 = pltpu.async_copy(
                emb_hbm.at[idx_v.at[rl, pl.ds(_G0, _G1)]],
                rows_v.at[pl.ds(_G0, _G1), :], sem_g)
            c0.wait()
            c1.wait()

            def red_body(j, acc):
                a0, a1 = acc
                for k in range(_UNROLL):
                    r = j * _UNROLL + k
                    a0 = a0 + rows_v[r, pl.ds(0, 16)]
                    a1 = a1 + rows_v[r, pl.ds(16, 16)]
                return a0, a1

            z = jnp.zeros((16,), jnp.float32)
            a0, a1 = lax.fori_loop(0, _H // _UNROLL, red_body, (z, z))
            out_v[rl, pl.ds(0, 16)] = a0
            out_v[rl, pl.ds(16, 16)] = a1
            return carry

        lax.fori_loop(0, _IB, row_body, 0)
        pltpu.sync_copy(out_v, out_hbm.at[pl.ds(base, _IB), :])
        return carry

    lax.fori_loop(0, _NBLK, blk_body, 0)


_emb_pool = functools.partial(
    pl.kernel,
    out_type=jax.ShapeDtypeStruct((_B, _D), jnp.float32),
    mesh=plsc.VectorSubcoreMesh(core_axis_name="c", subcore_axis_name="s"),
    compiler_params=pltpu.CompilerParams(use_tc_tiling_on_sc=False),
    scratch_types=[
        pltpu.VMEM((_IB, _H), jnp.int32),
        pltpu.VMEM((_H, _D), jnp.float32),
        pltpu.VMEM((_IB, _D), jnp.float32),
        pltpu.SemaphoreType.DMA,
    ],
)(_emb_pool_body)


@jax.jit
def kernel(x, action_emb):
    return _emb_pool(x, action_emb)


# speedup vs baseline: 36.5712x; 1.8174x over previous
import functools

import jax
import jax.numpy as jnp
from jax import lax
from jax.experimental import pallas as pl
from jax.experimental.pallas import tpu as pltpu
from jax.experimental.pallas import tpu_sc as plsc

_B, _H, _D = 16384, 200, 32
_NC, _NS = 2, 16
_NW = _NC * _NS
_RPW = _B // _NW
_K = 40
_NP = _H // _K
_OB = 32
_RUN = 8


def _emb_pool_body(x_hbm, emb_hbm, out_hbm, idx_v, acc_v, out_v, sem_g):
    wid = lax.axis_index("s") * _NC + lax.axis_index("c")
    row0 = wid * _RPW

    pltpu.sync_copy(x_hbm.at[pl.ds(row0, _RPW), :], idx_v)

    z = jnp.zeros((16,), jnp.float32)

    def zero_body(j, carry):
        for p in range(2):
            acc_v[p, j, pl.ds(0, 16)] = z
            acc_v[p, j, pl.ds(16, 16)] = z
        return carry

    lax.fori_loop(0, _K, zero_body, 0)

    def fire(r, b):
        for k in range(_NP):
            pltpu.async_copy(
                emb_hbm.at[idx_v.at[r, pl.ds(k * _K, _K)]],
                acc_v.at[b], sem_g, add=True)

    def drain(r, b):
        for k in range(_NP):
            pltpu.make_async_copy(
                emb_hbm.at[idx_v.at[r, pl.ds(k * _K, _K)]],
                acc_v.at[b], sem_g).wait()

    fire(0, 0)

    def row_body(r, carry):
        p = lax.rem(r, 2)

        @pl.when(r + 1 < _RPW)
        def _():
            fire(r + 1, 1 - p)

        drain(r, p)

        def red_body(j, acc):
            a0, a1, a2, a3 = acc
            for k in range(_RUN):
                jj = j * _RUN + k
                if k % 2 == 0:
                    a0 = a0 + acc_v[p, jj, pl.ds(0, 16)]
                    a1 = a1 + acc_v[p, jj, pl.ds(16, 16)]
                else:
                    a2 = a2 + acc_v[p, jj, pl.ds(0, 16)]
                    a3 = a3 + acc_v[p, jj, pl.ds(16, 16)]
                acc_v[p, jj, pl.ds(0, 16)] = z
                acc_v[p, jj, pl.ds(16, 16)] = z
            return a0, a1, a2, a3

        a0, a1, a2, a3 = lax.fori_loop(0, _K // _RUN, red_body, (z, z, z, z))
        rl = lax.rem(r, _OB)
        out_v[rl, pl.ds(0, 16)] = a0 + a2
        out_v[rl, pl.ds(16, 16)] = a1 + a3

        @pl.when(rl == _OB - 1)
        def _():
            pltpu.sync_copy(out_v, out_hbm.at[pl.ds(row0 + r - (_OB - 1), _OB), :])

        return carry

    lax.fori_loop(0, _RPW, row_body, 0)


_emb_pool = functools.partial(
    pl.kernel,
    out_type=jax.ShapeDtypeStruct((_B, _D), jnp.float32),
    mesh=plsc.VectorSubcoreMesh(core_axis_name="c", subcore_axis_name="s"),
    compiler_params=pltpu.CompilerParams(use_tc_tiling_on_sc=False),
    scratch_types=[
        pltpu.VMEM((_RPW, _H), jnp.int32),
        pltpu.VMEM((2, _K, _D), jnp.float32),
        pltpu.VMEM((_OB, _D), jnp.float32),
        pltpu.SemaphoreType.DMA,
    ],
)(_emb_pool_body)


@jax.jit
def kernel(x, action_emb):
    return _emb_pool(x, action_emb)


# speedup vs baseline: 51.3791x; 1.4049x over previous
import functools

import jax
import jax.numpy as jnp
from jax import lax
from jax.experimental import pallas as pl
from jax.experimental.pallas import tpu as pltpu
from jax.experimental.pallas import tpu_sc as plsc

_B, _H, _D = 16384, 200, 32
_NC, _NS = 2, 16
_NW = _NC * _NS
_RPW = _B // _NW
_K = 40
_NP = _H // _K
_OB = 32
_RUN = 8
_NBUF = 4


def _emb_pool_body(x_hbm, emb_hbm, out_hbm, idx_v, acc_v, out_v, sem_g):
    wid = lax.axis_index("s") * _NC + lax.axis_index("c")
    row0 = wid * _RPW

    pltpu.sync_copy(x_hbm.at[pl.ds(row0, _RPW), :], idx_v)

    z = jnp.zeros((16,), jnp.float32)

    def zero_body(j, carry):
        for p in range(_NBUF):
            acc_v[p, j, pl.ds(0, 16)] = z
            acc_v[p, j, pl.ds(16, 16)] = z
        return carry

    lax.fori_loop(0, _K, zero_body, 0)

    def fire(r, b):
        for k in range(_NP):
            pltpu.async_copy(
                emb_hbm.at[idx_v.at[r, pl.ds(k * _K, _K)]],
                acc_v.at[b], sem_g, add=True)

    def drain(r, b):
        for k in range(_NP):
            pltpu.make_async_copy(
                emb_hbm.at[idx_v.at[r, pl.ds(k * _K, _K)]],
                acc_v.at[b], sem_g).wait()

    for rr in range(_NBUF - 1):
        fire(rr, rr)

    def row_body(r, carry):
        p = lax.rem(r, _NBUF)

        @pl.when(r + _NBUF - 1 < _RPW)
        def _():
            fire(r + _NBUF - 1, lax.rem(r + _NBUF - 1, _NBUF))

        drain(r, p)

        def red_body(j, acc):
            a0, a1, a2, a3 = acc
            for k in range(_RUN):
                jj = j * _RUN + k
                if k % 2 == 0:
                    a0 = a0 + acc_v[p, jj, pl.ds(0, 16)]
                    a1 = a1 + acc_v[p, jj, pl.ds(16, 16)]
                else:
                    a2 = a2 + acc_v[p, jj, pl.ds(0, 16)]
                    a3 = a3 + acc_v[p, jj, pl.ds(16, 16)]
                acc_v[p, jj, pl.ds(0, 16)] = z
                acc_v[p, jj, pl.ds(16, 16)] = z
            return a0, a1, a2, a3

        a0, a1, a2, a3 = lax.fori_loop(0, _K // _RUN, red_body, (z, z, z, z))
        rl = lax.rem(r, _OB)
        out_v[rl, pl.ds(0, 16)] = a0 + a2
        out_v[rl, pl.ds(16, 16)] = a1 + a3

        @pl.when(rl == _OB - 1)
        def _():
            pltpu.sync_copy(out_v, out_hbm.at[pl.ds(row0 + r - (_OB - 1), _OB), :])

        return carry

    lax.fori_loop(0, _RPW, row_body, 0)


_emb_pool = functools.partial(
    pl.kernel,
    out_type=jax.ShapeDtypeStruct((_B, _D), jnp.float32),
    mesh=plsc.VectorSubcoreMesh(core_axis_name="c", subcore_axis_name="s"),
    compiler_params=pltpu.CompilerParams(use_tc_tiling_on_sc=False),
    scratch_types=[
        pltpu.VMEM((_RPW, _H), jnp.int32),
        pltpu.VMEM((_NBUF, _K, _D), jnp.float32),
        pltpu.VMEM((_OB, _D), jnp.float32),
        pltpu.SemaphoreType.DMA,
    ],
)(_emb_pool_body)


@jax.jit
def kernel(x, action_emb):
    return _emb_pool(x, action_emb)


# speedup vs baseline: 54.6709x; 1.0641x over previous
import functools

import jax
import jax.numpy as jnp
from jax import lax
from jax.experimental import pallas as pl
from jax.experimental.pallas import tpu as pltpu
from jax.experimental.pallas import tpu_sc as plsc

_B, _H, _D = 16384, 200, 32
_NC, _NS = 2, 16
_NW = _NC * _NS
_RPW = _B // _NW
_K = 40
_NP = _H // _K
_OB = 32
_RUN = 8
_NBUF = 8


def _emb_pool_body(x_hbm, emb_hbm, out_hbm, idx_v, acc_v, out_v, sem_g):
    wid = lax.axis_index("s") * _NC + lax.axis_index("c")
    row0 = wid * _RPW

    pltpu.sync_copy(x_hbm.at[pl.ds(row0, _RPW), :], idx_v)

    z = jnp.zeros((16,), jnp.float32)

    def zero_body(j, carry):
        for p in range(_NBUF):
            acc_v[p, j, pl.ds(0, 16)] = z
            acc_v[p, j, pl.ds(16, 16)] = z
        return carry

    lax.fori_loop(0, _K, zero_body, 0)

    def fire(r, b):
        for k in range(_NP):
            pltpu.async_copy(
                emb_hbm.at[idx_v.at[r, pl.ds(k * _K, _K)]],
                acc_v.at[b], sem_g, add=True)

    def drain(r, b):
        for k in range(_NP):
            pltpu.make_async_copy(
                emb_hbm.at[idx_v.at[r, pl.ds(k * _K, _K)]],
                acc_v.at[b], sem_g).wait()

    for rr in range(_NBUF - 1):
        fire(rr, rr)

    def row_body(r, carry):
        p = lax.rem(r, _NBUF)

        @pl.when(r + _NBUF - 1 < _RPW)
        def _():
            fire(r + _NBUF - 1, lax.rem(r + _NBUF - 1, _NBUF))

        drain(r, p)

        def red_body(j, acc):
            a0, a1, a2, a3 = acc
            for k in range(_RUN):
                jj = j * _RUN + k
                if k % 2 == 0:
                    a0 = a0 + acc_v[p, jj, pl.ds(0, 16)]
                    a1 = a1 + acc_v[p, jj, pl.ds(16, 16)]
                else:
                    a2 = a2 + acc_v[p, jj, pl.ds(0, 16)]
                    a3 = a3 + acc_v[p, jj, pl.ds(16, 16)]
                acc_v[p, jj, pl.ds(0, 16)] = z
                acc_v[p, jj, pl.ds(16, 16)] = z
            return a0, a1, a2, a3

        a0, a1, a2, a3 = lax.fori_loop(0, _K // _RUN, red_body, (z, z, z, z))
        rl = lax.rem(r, _OB)
        out_v[rl, pl.ds(0, 16)] = a0 + a2
        out_v[rl, pl.ds(16, 16)] = a1 + a3

        @pl.when(rl == _OB - 1)
        def _():
            pltpu.sync_copy(out_v, out_hbm.at[pl.ds(row0 + r - (_OB - 1), _OB), :])

        return carry

    lax.fori_loop(0, _RPW, row_body, 0)


_emb_pool = functools.partial(
    pl.kernel,
    out_type=jax.ShapeDtypeStruct((_B, _D), jnp.float32),
    mesh=plsc.VectorSubcoreMesh(core_axis_name="c", subcore_axis_name="s"),
    compiler_params=pltpu.CompilerParams(use_tc_tiling_on_sc=False),
    scratch_types=[
        pltpu.VMEM((_RPW, _H), jnp.int32),
        pltpu.VMEM((_NBUF, _K, _D), jnp.float32),
        pltpu.VMEM((_OB, _D), jnp.float32),
        pltpu.SemaphoreType.DMA,
    ],
)(_emb_pool_body)


@jax.jit
def kernel(x, action_emb):
    return _emb_pool(x, action_emb)
